# baseline (device time: 15540 ns/iter reference)
import os

import jax
import jax.numpy as jnp
from jax import lax
from jax.experimental import pallas as pl
from jax.experimental.pallas import tpu as pltpu

N_DEV = 4
EPS = 1e-5
N_PHASE = 4
N_CHUNK = 8
_NO_COMM = os.environ.get("KERNEL_NO_COMM", "0") == "1"


def kernel(x, gamma, beta):
    m, n_shard = x.shape
    n_global = n_shard * N_DEV
    mh = m // N_PHASE
    mc = m // N_CHUNK
    cpp = N_CHUNK // N_PHASE
    prs, pc = mh // 128, 128
    prc = mc // 128

    gamma2 = gamma.reshape(1, n_shard)
    beta2 = beta.reshape(1, n_shard)

    def body(x_hbm, g_hbm, b_hbm, out_hbm, xv, gv, bv, ov, comm_ref,
             in_sems, gb_sems, out_sems, send_sems, recv_sems):
        my = lax.axis_index("i")

        in_dmas = [
            pltpu.make_async_copy(
                x_hbm.at[pl.ds(c * mc, mc), :],
                xv.at[pl.ds(c * mc, mc), :],
                in_sems.at[c],
            )
            for c in range(N_CHUNK)
        ]
        for dma in in_dmas:
            dma.start()
        gdma = pltpu.make_async_copy(g_hbm, gv, gb_sems.at[0])
        bdma = pltpu.make_async_copy(b_hbm, bv, gb_sems.at[1])
        gdma.start()
        bdma.start()

        if not _NO_COMM:
            barrier_sem = pltpu.get_barrier_semaphore()
            for k in range(1, N_DEV):
                pl.semaphore_signal(
                    barrier_sem, inc=1,
                    device_id=(lax.rem(my + k, N_DEV),),
                    device_id_type=pl.DeviceIdType.MESH,
                )

        def onehots(rows, blocks):
            lane = lax.broadcasted_iota(jnp.int32, (rows, pc), 1)
            row = lax.broadcasted_iota(jnp.int32, (rows, pc), 0)
            mk = (lane == row % pc).astype(jnp.float32)
            sl = (
                lax.broadcasted_iota(jnp.int32, (rows, blocks), 1)
                == lax.broadcasted_iota(jnp.int32, (rows, blocks), 0) // pc
            ).astype(jnp.float32)
            return mk, sl

        mask_c, sel_c = onehots(mc, prc)
        mask_h, sel_h = onehots(mh, prs)

        def pack(s):
            return lax.dot_general(
                sel_c, s * mask_c, (((0,), (0,)), ((), ())),
                preferred_element_type=jnp.float32,
            )

        def unpack(t):
            u = lax.dot_general(
                sel_h, t, (((1,), (0,)), ((), ())),
                preferred_element_type=jnp.float32,
            )
            return jnp.sum(u * mask_h, axis=1, keepdims=True)

        def partial_sums(p):
            for ci in range(cpp):
                c = p * cpp + ci
                in_dmas[c].wait()
                xf = xv[pl.ds(c * mc, mc), :]
                s1 = jnp.sum(xf, axis=1, keepdims=True)
                s2 = jnp.sum(xf * xf, axis=1, keepdims=True)
                comm_ref[p, 0, 0, pl.ds(ci * prc, prc)] = pack(s1)
                comm_ref[p, 0, 1, pl.ds(ci * prc, prc)] = pack(s2)

        def start_sends(p):
            rdmas = []
            for k in range(1, N_DEV):
                rdma = pltpu.make_async_remote_copy(
                    src_ref=comm_ref.at[p, 0],
                    dst_ref=comm_ref.at[p, k],
                    send_sem=send_sems.at[p, k - 1],
                    recv_sem=recv_sems.at[p, k - 1],
                    device_id=(lax.rem(my + k, N_DEV),),
                    device_id_type=pl.DeviceIdType.MESH,
                )
                rdma.start()
                rdmas.append(rdma)
            return rdmas

        def normalize(p, rdmas, gb, bb):
            for rdma in rdmas:
                rdma.wait_recv()
            if _NO_COMM:
                tot1 = comm_ref[p, 0, 0] * 4.0
                tot2 = comm_ref[p, 0, 1] * 4.0
            else:
                tot1 = (comm_ref[p, 0, 0] + comm_ref[p, 1, 0]
                        + comm_ref[p, 2, 0] + comm_ref[p, 3, 0])
                tot2 = (comm_ref[p, 0, 1] + comm_ref[p, 1, 1]
                        + comm_ref[p, 2, 1] + comm_ref[p, 3, 1])
            inv_n = 1.0 / n_global
            mean = unpack(tot1) * inv_n
            var = unpack(tot2) * inv_n - mean * mean
            rstd = lax.rsqrt(var + EPS)
            xb = xv[pl.ds(p * mh, mh), :].astype(jnp.bfloat16)
            ov[pl.ds(p * mh, mh), :] = (
                (xb - mean.astype(jnp.bfloat16)) * rstd.astype(jnp.bfloat16)
                * gb + bb
            )
            odma = pltpu.make_async_copy(
                ov.at[pl.ds(p * mh, mh), :],
                out_hbm.at[pl.ds(p * mh, mh), :],
                out_sems.at[p],
            )
            odma.start()
            return odma

        rdmas = []
        for p in range(N_PHASE):
            partial_sums(p)
            if not _NO_COMM:
                if p == 0:
                    pl.semaphore_wait(barrier_sem, N_DEV - 1)
                rdmas.append(start_sends(p))
            else:
                rdmas.append([])

        gdma.wait()
        bdma.wait()
        gb = gv[:, :].astype(jnp.bfloat16)
        bb = bv[:, :].astype(jnp.bfloat16)

        odmas = [normalize(p, rdmas[p], gb, bb) for p in range(N_PHASE)]
        for odma in odmas:
            odma.wait()
        for group in rdmas:
            for rdma in group:
                rdma.wait_send()

    return pl.pallas_call(
        body,
        out_shape=jax.ShapeDtypeStruct((m, n_shard), jnp.bfloat16),
        in_specs=[
            pl.BlockSpec(memory_space=pltpu.MemorySpace.HBM),
            pl.BlockSpec(memory_space=pltpu.MemorySpace.HBM),
            pl.BlockSpec(memory_space=pltpu.MemorySpace.HBM),
        ],
        out_specs=pl.BlockSpec(memory_space=pltpu.MemorySpace.HBM),
        scratch_shapes=[
            pltpu.VMEM((m, n_shard), jnp.float32),
            pltpu.VMEM((1, n_shard), jnp.float32),
            pltpu.VMEM((1, n_shard), jnp.float32),
            pltpu.VMEM((m, n_shard), jnp.bfloat16),
            pltpu.VMEM((N_PHASE, N_DEV, 2, prs, pc), jnp.float32),
            pltpu.SemaphoreType.DMA((N_CHUNK,)),
            pltpu.SemaphoreType.DMA((2,)),
            pltpu.SemaphoreType.DMA((N_PHASE,)),
            pltpu.SemaphoreType.DMA((N_PHASE, N_DEV - 1)),
            pltpu.SemaphoreType.DMA((N_PHASE, N_DEV - 1)),
        ],
        compiler_params=(
            pltpu.CompilerParams()
            if _NO_COMM
            else pltpu.CompilerParams(collective_id=0)
        ),
    )(x, gamma2, beta2)


# device time: 11384 ns/iter; 1.3651x vs baseline; 1.3651x over previous
import os

import jax
import jax.numpy as jnp
from jax import lax
from jax.experimental import pallas as pl
from jax.experimental.pallas import tpu as pltpu

N_DEV = 4
EPS = 1e-5
N_PHASE = 4
N_CHUNK = 8
_NO_COMM = os.environ.get("KERNEL_NO_COMM", "0") == "1"


def kernel(x, gamma, beta):
    m, n_shard = x.shape
    n_global = n_shard * N_DEV
    mh = m // N_PHASE
    mc = m // N_CHUNK
    cpp = N_CHUNK // N_PHASE
    prs, pc = mh // 128, 128
    prc = mc // 128

    gamma2 = gamma.reshape(1, n_shard)
    beta2 = beta.reshape(1, n_shard)
    x = pltpu.with_memory_space_constraint(x, pltpu.MemorySpace.HBM)
    gamma2 = pltpu.with_memory_space_constraint(gamma2, pltpu.MemorySpace.HBM)
    beta2 = pltpu.with_memory_space_constraint(beta2, pltpu.MemorySpace.HBM)

    def body(x_hbm, g_hbm, b_hbm, out_hbm, xv, gv, bv, ov, comm_ref,
             in_sems, gb_sems, out_sems, send_sems, recv_sems):
        my = lax.axis_index("i")

        in_dmas = [
            pltpu.make_async_copy(
                x_hbm.at[pl.ds(c * mc, mc), :],
                xv.at[pl.ds(c * mc, mc), :],
                in_sems.at[c],
            )
            for c in range(N_CHUNK)
        ]
        for dma in in_dmas:
            dma.start()
        gdma = pltpu.make_async_copy(g_hbm, gv, gb_sems.at[0])
        bdma = pltpu.make_async_copy(b_hbm, bv, gb_sems.at[1])
        gdma.start()
        bdma.start()

        if not _NO_COMM:
            barrier_sem = pltpu.get_barrier_semaphore()
            for k in range(1, N_DEV):
                pl.semaphore_signal(
                    barrier_sem, inc=1,
                    device_id=(lax.rem(my + k, N_DEV),),
                    device_id_type=pl.DeviceIdType.MESH,
                )

        def onehots(rows, blocks):
            lane = lax.broadcasted_iota(jnp.int32, (rows, pc), 1)
            row = lax.broadcasted_iota(jnp.int32, (rows, pc), 0)
            mk = (lane == row % pc).astype(jnp.float32)
            sl = (
                lax.broadcasted_iota(jnp.int32, (rows, blocks), 1)
                == lax.broadcasted_iota(jnp.int32, (rows, blocks), 0) // pc
            ).astype(jnp.float32)
            return mk, sl

        mask_c, sel_c = onehots(mc, prc)
        mask_h, sel_h = onehots(mh, prs)

        def pack(s):
            return lax.dot_general(
                sel_c, s * mask_c, (((0,), (0,)), ((), ())),
                preferred_element_type=jnp.float32,
            )

        def unpack(t):
            u = lax.dot_general(
                sel_h, t, (((1,), (0,)), ((), ())),
                preferred_element_type=jnp.float32,
            )
            return jnp.sum(u * mask_h, axis=1, keepdims=True)

        def partial_sums(p):
            for ci in range(cpp):
                c = p * cpp + ci
                in_dmas[c].wait()
                xf = xv[pl.ds(c * mc, mc), :]
                s1 = jnp.sum(xf, axis=1, keepdims=True)
                s2 = jnp.sum(xf * xf, axis=1, keepdims=True)
                comm_ref[p, 0, 0, pl.ds(ci * prc, prc)] = pack(s1)
                comm_ref[p, 0, 1, pl.ds(ci * prc, prc)] = pack(s2)

        def start_sends(p):
            rdmas = []
            for k in range(1, N_DEV):
                rdma = pltpu.make_async_remote_copy(
                    src_ref=comm_ref.at[p, 0],
                    dst_ref=comm_ref.at[p, k],
                    send_sem=send_sems.at[p, k - 1],
                    recv_sem=recv_sems.at[p, k - 1],
                    device_id=(lax.rem(my + k, N_DEV),),
                    device_id_type=pl.DeviceIdType.MESH,
                )
                rdma.start()
                rdmas.append(rdma)
            return rdmas

        def normalize(p, rdmas, gb, bb):
            for rdma in rdmas:
                rdma.wait_recv()
            if _NO_COMM:
                tot1 = comm_ref[p, 0, 0] * 4.0
                tot2 = comm_ref[p, 0, 1] * 4.0
            else:
                tot1 = (comm_ref[p, 0, 0] + comm_ref[p, 1, 0]
                        + comm_ref[p, 2, 0] + comm_ref[p, 3, 0])
                tot2 = (comm_ref[p, 0, 1] + comm_ref[p, 1, 1]
                        + comm_ref[p, 2, 1] + comm_ref[p, 3, 1])
            inv_n = 1.0 / n_global
            mean = unpack(tot1) * inv_n
            var = unpack(tot2) * inv_n - mean * mean
            rstd = lax.rsqrt(var + EPS)
            xb = xv[pl.ds(p * mh, mh), :].astype(jnp.bfloat16)
            ov[pl.ds(p * mh, mh), :] = (
                (xb - mean.astype(jnp.bfloat16)) * rstd.astype(jnp.bfloat16)
                * gb + bb
            )
            odma = pltpu.make_async_copy(
                ov.at[pl.ds(p * mh, mh), :],
                out_hbm.at[pl.ds(p * mh, mh), :],
                out_sems.at[p],
            )
            odma.start()
            return odma

        rdmas = []
        for p in range(N_PHASE):
            partial_sums(p)
            if not _NO_COMM:
                if p == 0:
                    pl.semaphore_wait(barrier_sem, N_DEV - 1)
                rdmas.append(start_sends(p))
            else:
                rdmas.append([])

        gdma.wait()
        bdma.wait()
        gb = gv[:, :].astype(jnp.bfloat16)
        bb = bv[:, :].astype(jnp.bfloat16)

        odmas = [normalize(p, rdmas[p], gb, bb) for p in range(N_PHASE)]
        for odma in odmas:
            odma.wait()
        for group in rdmas:
            for rdma in group:
                rdma.wait_send()

    return pl.pallas_call(
        body,
        out_shape=jax.ShapeDtypeStruct((m, n_shard), jnp.bfloat16),
        in_specs=[
            pl.BlockSpec(memory_space=pltpu.MemorySpace.HBM),
            pl.BlockSpec(memory_space=pltpu.MemorySpace.HBM),
            pl.BlockSpec(memory_space=pltpu.MemorySpace.HBM),
        ],
        out_specs=pl.BlockSpec(memory_space=pltpu.MemorySpace.HBM),
        scratch_shapes=[
            pltpu.VMEM((m, n_shard), jnp.float32),
            pltpu.VMEM((1, n_shard), jnp.float32),
            pltpu.VMEM((1, n_shard), jnp.float32),
            pltpu.VMEM((m, n_shard), jnp.bfloat16),
            pltpu.VMEM((N_PHASE, N_DEV, 2, prs, pc), jnp.float32),
            pltpu.SemaphoreType.DMA((N_CHUNK,)),
            pltpu.SemaphoreType.DMA((2,)),
            pltpu.SemaphoreType.DMA((N_PHASE,)),
            pltpu.SemaphoreType.DMA((N_PHASE, N_DEV - 1)),
            pltpu.SemaphoreType.DMA((N_PHASE, N_DEV - 1)),
        ],
        compiler_params=(
            pltpu.CompilerParams()
            if _NO_COMM
            else pltpu.CompilerParams(collective_id=0)
        ),
    )(x, gamma2, beta2)
